# Initial kernel scaffold; baseline (speedup 1.0000x reference)
import jax
import jax.numpy as jnp
from jax.experimental import pallas as pl


def kernel(x, embeddings):
    raise NotImplementedError("write your pallas kernel here")


# speedup vs baseline: 1.2124x; 1.2124x over previous
import functools

import jax
import jax.numpy as jnp
from jax import lax
from jax.experimental import pallas as pl
from jax.experimental.pallas import tpu as pltpu
from jax.experimental.pallas import tpu_sc as plsc

N = 4096
K = 8192
D = 32

BN = 512
BK = 2048


def _argmin_body(x_ref, et_ref, idx_ref):
    x_blk = x_ref[...]
    x_sq = jnp.sum(x_blk * x_blk, axis=1, keepdims=True)
    run_min = jnp.full((BN, 1), jnp.inf, dtype=jnp.float32)
    run_idx = jnp.zeros((BN, 1), dtype=jnp.int32)
    for kc in range(K // BK):
        et_c = et_ref[:, kc * BK:(kc + 1) * BK]
        e_sq = jnp.sum(et_c * et_c, axis=0, keepdims=True)
        m = jnp.dot(x_blk, et_c, preferred_element_type=jnp.float32)
        d = (x_sq - 2.0 * m) + e_sq
        c_min = jnp.min(d, axis=1, keepdims=True)
        iota = lax.broadcasted_iota(jnp.int32, (BN, BK), 1) + kc * BK
        c_idx = jnp.min(jnp.where(d == c_min, iota, K), axis=1, keepdims=True)
        better = c_min < run_min
        run_idx = jnp.where(better, c_idx, run_idx)
        run_min = jnp.where(better, c_min, run_min)
    idx_ref[...] = run_idx


_argmin_call = pl.pallas_call(
    _argmin_body,
    grid=(N // BN,),
    in_specs=[
        pl.BlockSpec((BN, D), lambda i: (i, 0)),
        pl.BlockSpec((D, K), lambda i: (0, 0)),
    ],
    out_specs=pl.BlockSpec((BN, 1), lambda i: (i, 0)),
    out_shape=jax.ShapeDtypeStruct((N, 1), jnp.int32),
)


_NC, _NS = 2, 16
_NW = _NC * _NS
_BPW = N // _NW


@functools.cache
def _make_gather_call():
    @functools.partial(
        pl.kernel,
        mesh=plsc.VectorSubcoreMesh(core_axis_name="c", subcore_axis_name="s"),
        out_type=jax.ShapeDtypeStruct((N, D), jnp.float32),
        scratch_types=[
            pltpu.VMEM((_BPW,), jnp.int32),
            pltpu.VMEM((_BPW, D), jnp.float32),
            pltpu.SemaphoreType.DMA,
        ],
        compiler_params=pltpu.CompilerParams(use_tc_tiling_on_sc=False),
    )
    def _gather_call(table_hbm, idx_hbm, out_hbm, idx_v, rows_v, sem):
        wid = lax.axis_index("s") * _NC + lax.axis_index("c")
        base = wid * _BPW
        pltpu.sync_copy(idx_hbm.at[pl.ds(base, _BPW)], idx_v)
        pltpu.async_copy(table_hbm.at[idx_v], rows_v, sem).wait()
        pltpu.sync_copy(rows_v, out_hbm.at[pl.ds(base, _BPW)])

    return _gather_call


def kernel(x, embeddings):
    x_ = x.reshape(N, D)
    et = embeddings.T
    idx = _argmin_call(x_, et).reshape(N)
    quantized = _make_gather_call()(embeddings, idx).reshape(-1)
    z_hat = x + quantized - lax.stop_gradient(x)
    return (x, quantized, z_hat, idx)


# speedup vs baseline: 1.3023x; 1.0741x over previous
import functools

import jax
import jax.numpy as jnp
from jax import lax
from jax.experimental import pallas as pl
from jax.experimental.pallas import tpu as pltpu
from jax.experimental.pallas import tpu_sc as plsc

N = 4096
K = 8192
D = 32

BN = 512
BK = 2048


def _argmin_body(xt_ref, e_ref, kio_ref, idx_ref):
    xt = xt_ref[...]
    x2t = xt * -2.0
    x_sq = jnp.sum(xt * xt, axis=0, keepdims=True)
    run_min = jnp.full((1, BN), jnp.inf, dtype=jnp.float32)
    run_idx = jnp.full((1, BN), float(K), dtype=jnp.float32)
    for kc in range(K // BK):
        e_c = e_ref[kc * BK:(kc + 1) * BK, :]
        e_sq = jnp.sum(e_c * e_c, axis=1, keepdims=True)
        m2 = jnp.dot(e_c, x2t, preferred_element_type=jnp.float32)
        d = (x_sq + m2) + e_sq
        c_min = jnp.min(d, axis=0, keepdims=True)
        kio = kio_ref[kc * BK:(kc + 1) * BK, :]
        c_idx = jnp.min(jnp.where(d == c_min, kio, float(K)),
                        axis=0, keepdims=True)
        better = c_min < run_min
        run_idx = jnp.where(better, c_idx, run_idx)
        run_min = jnp.where(better, c_min, run_min)
    idx_ref[...] = run_idx.astype(jnp.int32)[None]


_argmin_call = pl.pallas_call(
    _argmin_body,
    grid=(N // BN,),
    in_specs=[
        pl.BlockSpec((D, BN), lambda i: (0, i)),
        pl.BlockSpec((K, D), lambda i: (0, 0)),
        pl.BlockSpec((K, 1), lambda i: (0, 0)),
    ],
    out_specs=pl.BlockSpec((1, 1, BN), lambda i: (i, 0, 0)),
    out_shape=jax.ShapeDtypeStruct((N // BN, 1, BN), jnp.int32),
)


_NC, _NS = 2, 16
_NW = _NC * _NS
_BPW = N // _NW


@functools.cache
def _make_gather_call():
    @functools.partial(
        pl.kernel,
        mesh=plsc.VectorSubcoreMesh(core_axis_name="c", subcore_axis_name="s"),
        out_type=jax.ShapeDtypeStruct((N, D), jnp.float32),
        scratch_types=[
            pltpu.VMEM((_BPW,), jnp.int32),
            pltpu.VMEM((_BPW, D), jnp.float32),
            pltpu.SemaphoreType.DMA,
        ],
        compiler_params=pltpu.CompilerParams(use_tc_tiling_on_sc=False),
    )
    def _gather_call(table_hbm, idx_hbm, out_hbm, idx_v, rows_v, sem):
        wid = lax.axis_index("s") * _NC + lax.axis_index("c")
        base = wid * _BPW
        pltpu.sync_copy(idx_hbm.at[pl.ds(base, _BPW)], idx_v)
        pltpu.async_copy(table_hbm.at[idx_v], rows_v, sem).wait()
        pltpu.sync_copy(rows_v, out_hbm.at[pl.ds(base, _BPW)])

    return _gather_call


def kernel(x, embeddings):
    xt = x.reshape(N, D).T
    kio = lax.broadcasted_iota(jnp.float32, (K, 1), 0)
    idx = _argmin_call(xt, embeddings, kio).reshape(N)
    quantized = _make_gather_call()(embeddings, idx).reshape(-1)
    z_hat = x + quantized - lax.stop_gradient(x)
    return (x, quantized, z_hat, idx)


# speedup vs baseline: 1.3464x; 1.0338x over previous
import functools

import jax
import jax.numpy as jnp
from jax import lax
from jax.experimental import pallas as pl
from jax.experimental.pallas import tpu as pltpu
from jax.experimental.pallas import tpu_sc as plsc

N = 4096
K = 8192
D = 32

BN = 512
BK = 2048


AR = 8


def _argmin_body(xt_ref, e_ref, kio_ref, idx_ref):
    xt = xt_ref[...]
    x2t = xt * -2.0
    x_sq = jnp.sum(xt * xt, axis=0, keepdims=True)
    run_min = jnp.full((AR, BN), jnp.inf, dtype=jnp.float32)
    run_idx = jnp.full((AR, BN), float(K), dtype=jnp.float32)
    for kc in range(K // BK):
        e_c = e_ref[kc * BK:(kc + 1) * BK, :]
        e_sq = jnp.sum(e_c * e_c, axis=1, keepdims=True)
        m2 = jnp.dot(e_c, x2t, preferred_element_type=jnp.float32)
        m3 = m2.reshape(BK // AR, AR, BN)
        e3 = e_sq.reshape(BK // AR, AR, 1)
        k3 = kio_ref[kc * BK:(kc + 1) * BK, :].reshape(BK // AR, AR, 1)
        for r in range(BK // AR):
            v = (x_sq + m3[r]) + e3[r]
            mask = v < run_min
            run_idx = jnp.where(mask, k3[r], run_idx)
            run_min = jnp.minimum(v, run_min)
    rows = AR
    while rows > 1:
        h = rows // 2
        a_min, b_min = run_min[:h], run_min[h:rows]
        a_idx, b_idx = run_idx[:h], run_idx[h:rows]
        take_b = (b_min < a_min) | ((b_min == a_min) & (b_idx < a_idx))
        run_min = jnp.where(take_b, b_min, a_min)
        run_idx = jnp.where(take_b, b_idx, a_idx)
        rows = h
    idx_ref[...] = run_idx.astype(jnp.int32)[None]


_argmin_call = pl.pallas_call(
    _argmin_body,
    grid=(N // BN,),
    in_specs=[
        pl.BlockSpec((D, BN), lambda i: (0, i)),
        pl.BlockSpec((K, D), lambda i: (0, 0)),
        pl.BlockSpec((K, 1), lambda i: (0, 0)),
    ],
    out_specs=pl.BlockSpec((1, 1, BN), lambda i: (i, 0, 0)),
    out_shape=jax.ShapeDtypeStruct((N // BN, 1, BN), jnp.int32),
    compiler_params=pltpu.CompilerParams(dimension_semantics=("parallel",)),
)


_NC, _NS = 2, 16
_NW = _NC * _NS
_BPW = N // _NW


@functools.cache
def _make_gather_call():
    @functools.partial(
        pl.kernel,
        mesh=plsc.VectorSubcoreMesh(core_axis_name="c", subcore_axis_name="s"),
        out_type=jax.ShapeDtypeStruct((N, D), jnp.float32),
        scratch_types=[
            pltpu.VMEM((_BPW,), jnp.int32),
            pltpu.VMEM((_BPW, D), jnp.float32),
            pltpu.SemaphoreType.DMA,
        ],
        compiler_params=pltpu.CompilerParams(use_tc_tiling_on_sc=False),
    )
    def _gather_call(table_hbm, idx_hbm, out_hbm, idx_v, rows_v, sem):
        wid = lax.axis_index("s") * _NC + lax.axis_index("c")
        base = wid * _BPW
        pltpu.sync_copy(idx_hbm.at[pl.ds(base, _BPW)], idx_v)
        pltpu.async_copy(table_hbm.at[idx_v], rows_v, sem).wait()
        pltpu.sync_copy(rows_v, out_hbm.at[pl.ds(base, _BPW)])

    return _gather_call


def kernel(x, embeddings):
    xt = x.reshape(N, D).T
    kio = lax.broadcasted_iota(jnp.float32, (K, 1), 0)
    idx = _argmin_call(xt, embeddings, kio).reshape(N)
    quantized = _make_gather_call()(embeddings, idx).reshape(-1)
    z_hat = x + quantized - lax.stop_gradient(x)
    return (x, quantized, z_hat, idx)


# speedup vs baseline: 1.5034x; 1.1166x over previous
import functools

import jax
import jax.numpy as jnp
from jax import lax
from jax.experimental import pallas as pl
from jax.experimental.pallas import tpu as pltpu
from jax.experimental.pallas import tpu_sc as plsc

N = 4096
K = 8192
D = 32

BN = 512
BK = 2048


AR = 8


def _argmin_body(xt_ref, e_ref, kio_ref, idx_ref, esq_ref):
    @pl.when(pl.program_id(0) == 0)
    def _():
        e_all = e_ref[...]
        esq_ref[...] = jnp.sum(e_all * e_all, axis=1, keepdims=True)

    xt = xt_ref[...]
    x2t = xt * -2.0
    x_sq = jnp.broadcast_to(
        jnp.sum(xt * xt, axis=0, keepdims=True), (AR, BN))
    run_min = jnp.full((AR, BN), jnp.inf, dtype=jnp.float32)
    run_idx = jnp.full((AR, BN), float(K), dtype=jnp.float32)
    for kc in range(K // BK):
        e_c = e_ref[kc * BK:(kc + 1) * BK, :]
        m2 = jnp.dot(e_c, x2t, preferred_element_type=jnp.float32)
        m3 = m2.reshape(BK // AR, AR, BN)
        e3 = esq_ref[kc * BK:(kc + 1) * BK, :].reshape(BK // AR, AR, 1)
        k3 = kio_ref[kc * BK:(kc + 1) * BK, :].reshape(BK // AR, AR, 1)
        for r in range(BK // AR):
            v = (x_sq + m3[r]) + e3[r]
            mask = v < run_min
            run_idx = jnp.where(mask, k3[r], run_idx)
            run_min = jnp.minimum(v, run_min)
    rows = AR
    while rows > 1:
        h = rows // 2
        a_min, b_min = run_min[:h], run_min[h:rows]
        a_idx, b_idx = run_idx[:h], run_idx[h:rows]
        take_b = (b_min < a_min) | ((b_min == a_min) & (b_idx < a_idx))
        run_min = jnp.where(take_b, b_min, a_min)
        run_idx = jnp.where(take_b, b_idx, a_idx)
        rows = h
    idx_ref[...] = run_idx.astype(jnp.int32)[None]


_argmin_call = pl.pallas_call(
    _argmin_body,
    grid=(N // BN,),
    in_specs=[
        pl.BlockSpec((D, BN), lambda i: (0, i)),
        pl.BlockSpec((K, D), lambda i: (0, 0)),
        pl.BlockSpec((K, 1), lambda i: (0, 0)),
    ],
    out_specs=pl.BlockSpec((1, 1, BN), lambda i: (i, 0, 0)),
    out_shape=jax.ShapeDtypeStruct((N // BN, 1, BN), jnp.int32),
    scratch_shapes=[pltpu.VMEM((K, 1), jnp.float32)],
)


_NC, _NS = 2, 16
_NW = _NC * _NS
_BPW = N // _NW


@functools.cache
def _make_gather_call():
    @functools.partial(
        pl.kernel,
        mesh=plsc.VectorSubcoreMesh(core_axis_name="c", subcore_axis_name="s"),
        out_type=jax.ShapeDtypeStruct((N, D), jnp.float32),
        scratch_types=[
            pltpu.VMEM((_BPW,), jnp.int32),
            pltpu.VMEM((_BPW, D), jnp.float32),
            pltpu.SemaphoreType.DMA,
        ],
        compiler_params=pltpu.CompilerParams(use_tc_tiling_on_sc=False),
    )
    def _gather_call(table_hbm, idx_hbm, out_hbm, idx_v, rows_v, sem):
        wid = lax.axis_index("s") * _NC + lax.axis_index("c")
        base = wid * _BPW
        pltpu.sync_copy(idx_hbm.at[pl.ds(base, _BPW)], idx_v)
        pltpu.async_copy(table_hbm.at[idx_v], rows_v, sem).wait()
        pltpu.sync_copy(rows_v, out_hbm.at[pl.ds(base, _BPW)])

    return _gather_call


def kernel(x, embeddings):
    xt = x.reshape(N, D).T
    kio = lax.broadcasted_iota(jnp.float32, (K, 1), 0)
    idx = _argmin_call(xt, embeddings, kio).reshape(N)
    quantized = _make_gather_call()(embeddings, idx).reshape(-1)
    z_hat = x + quantized - lax.stop_gradient(x)
    return (x, quantized, z_hat, idx)
